# baseline (speedup 1.0000x reference)
import math

import jax
import jax.numpy as jnp
from jax.experimental import pallas as pl

_N = 10000
_D = 128
_H = 8
_OC = 8
_C = 40
_T = 5.0
_K = 10
_NP = 10240
_BD = 256
_BA = 128
_F32 = jnp.float32


def _dinv_kernel(m_ref, dinv_ref):
    m = m_ref[...].astype(_F32)
    deg = jnp.sum(m, axis=1, keepdims=True)
    dinv_ref[...] = jnp.where(deg > 0, jax.lax.rsqrt(deg), 0.0)


def _td_step_kernel(coef, m_ref, dinv_full_ref, dinv_blk_ref, cur_ref,
                    acc_ref, cur_out_ref, acc_out_ref):
    m = m_ref[...]
    v = cur_ref[...] * dinv_full_ref[...]
    vh = v.astype(jnp.bfloat16)
    vl = (v - vh.astype(_F32)).astype(jnp.bfloat16)
    y = (jnp.dot(m, vh, preferred_element_type=_F32)
         + jnp.dot(m, vl, preferred_element_type=_F32))
    cur_new = y * dinv_blk_ref[...]
    cur_out_ref[...] = cur_new
    acc_out_ref[...] = acc_ref[...] + coef * cur_new


def _proj1_kernel(acc_ref, w1_ref, smat_ref, dmat_ref,
                  h1_ref, als_ref, ald_ref):
    h1 = jnp.dot(acc_ref[...], w1_ref[...], preferred_element_type=_F32)
    h1_ref[...] = h1
    als_ref[...] = jnp.dot(h1, smat_ref[...], preferred_element_type=_F32)
    ald_ref[...] = jnp.dot(h1, dmat_ref[...], preferred_element_type=_F32)


def _gat1_kernel(m_ref, alst_ref, ald_ref, h1_ref, b1_ref, w2_ref,
                 s2_ref, d2_ref, h2_ref, als2_ref, ald2_ref):
    m = m_ref[...].astype(_F32)
    alst = alst_ref[...]
    ald = ald_ref[...]
    h1 = h1_ref[...]
    esp = jnp.exp(alst)
    esn = jnp.exp(0.2 * alst)
    edp = jnp.exp(ald)
    edn = jnp.exp(0.2 * ald)
    outs = []
    for h in range(_H):
        p = esp[h:h + 1, :] * edp[:, h:h + 1]
        n = esn[h:h + 1, :] * edn[:, h:h + 1]
        w = jnp.maximum(p, n) * m
        den = jnp.sum(w, axis=1, keepdims=True)
        num = jnp.dot(w, h1[:, h * _OC:(h + 1) * _OC],
                      preferred_element_type=_F32)
        outs.append(num / (den + 1e-16))
    z = jnp.concatenate(outs, axis=1) + b1_ref[...]
    z = jnp.where(z > 0, z, jnp.exp(jnp.minimum(z, 0.0)) - 1.0)
    h2 = jnp.dot(z, w2_ref[...], preferred_element_type=_F32)
    h2_ref[...] = h2
    als2_ref[...] = jnp.dot(h2, s2_ref[...], preferred_element_type=_F32)
    ald2_ref[...] = jnp.dot(h2, d2_ref[...], preferred_element_type=_F32)


def _gat2_kernel(m_ref, als2t_ref, ald2_ref, h2_ref, b2_ref, out_ref):
    m = m_ref[...].astype(_F32)
    als2t = als2t_ref[...]
    ald2 = ald2_ref[...]
    p = jnp.exp(als2t) * jnp.exp(ald2)
    n = jnp.exp(0.2 * als2t) * jnp.exp(0.2 * ald2)
    w = jnp.maximum(p, n) * m
    den = jnp.sum(w, axis=1, keepdims=True)
    num = jnp.dot(w, h2_ref[...], preferred_element_type=_F32)
    o = num / (den + 1e-16) + b2_ref[...]
    mx = jnp.max(o, axis=1, keepdims=True)
    l = o - mx
    lse = jnp.log(jnp.sum(jnp.exp(l), axis=1, keepdims=True))
    out_ref[...] = l - lse


def kernel(x, edge_index, W1, att_src1, att_dst1, b1, W2, att_src2,
           att_dst2, b2):
    loops = jnp.arange(_N, dtype=jnp.int32)
    src = jnp.concatenate([edge_index[0].astype(jnp.int32), loops])
    dst = jnp.concatenate([edge_index[1].astype(jnp.int32), loops])
    flat = dst * _NP + src
    m = jnp.zeros((_NP * _NP,), jnp.bfloat16).at[flat].add(
        jnp.bfloat16(1)).reshape(_NP, _NP)

    xp = jnp.pad(x, ((0, _NP - _N), (0, 0)))

    g = _NP // _BD
    dinv = pl.pallas_call(
        _dinv_kernel,
        grid=(g,),
        in_specs=[pl.BlockSpec((_BD, _NP), lambda i: (i, 0))],
        out_specs=pl.BlockSpec((_BD, 1), lambda i: (i, 0)),
        out_shape=jax.ShapeDtypeStruct((_NP, 1), _F32),
    )(m)

    coef = math.exp(-_T)
    acc = coef * xp
    cur = xp
    for k in range(1, _K + 1):
        coef = coef * _T / k
        step = pl.pallas_call(
            lambda mr, df, db, cr, ar, co, ao, _c=coef: _td_step_kernel(
                _c, mr, df, db, cr, ar, co, ao),
            grid=(g,),
            in_specs=[
                pl.BlockSpec((_BD, _NP), lambda i: (i, 0)),
                pl.BlockSpec((_NP, 1), lambda i: (0, 0)),
                pl.BlockSpec((_BD, 1), lambda i: (i, 0)),
                pl.BlockSpec((_NP, _D), lambda i: (0, 0)),
                pl.BlockSpec((_BD, _D), lambda i: (i, 0)),
            ],
            out_specs=[
                pl.BlockSpec((_BD, _D), lambda i: (i, 0)),
                pl.BlockSpec((_BD, _D), lambda i: (i, 0)),
            ],
            out_shape=[
                jax.ShapeDtypeStruct((_NP, _D), _F32),
                jax.ShapeDtypeStruct((_NP, _D), _F32),
            ],
        )
        cur, acc = step(m, dinv, dinv, cur, acc)

    smat = jnp.einsum('ho,hk->hok', att_src1,
                      jnp.eye(_H, dtype=_F32)).reshape(_H * _OC, _H)
    dmat = jnp.einsum('ho,hk->hok', att_dst1,
                      jnp.eye(_H, dtype=_F32)).reshape(_H * _OC, _H)

    h1, als, ald = pl.pallas_call(
        _proj1_kernel,
        grid=(g,),
        in_specs=[
            pl.BlockSpec((_BD, _D), lambda i: (i, 0)),
            pl.BlockSpec((_D, _H * _OC), lambda i: (0, 0)),
            pl.BlockSpec((_H * _OC, _H), lambda i: (0, 0)),
            pl.BlockSpec((_H * _OC, _H), lambda i: (0, 0)),
        ],
        out_specs=[
            pl.BlockSpec((_BD, _H * _OC), lambda i: (i, 0)),
            pl.BlockSpec((_BD, _H), lambda i: (i, 0)),
            pl.BlockSpec((_BD, _H), lambda i: (i, 0)),
        ],
        out_shape=[
            jax.ShapeDtypeStruct((_NP, _H * _OC), _F32),
            jax.ShapeDtypeStruct((_NP, _H), _F32),
            jax.ShapeDtypeStruct((_NP, _H), _F32),
        ],
    )(acc, W1, smat, dmat)

    ga = _NP // _BA
    h2, als2, ald2 = pl.pallas_call(
        _gat1_kernel,
        grid=(ga,),
        in_specs=[
            pl.BlockSpec((_BA, _NP), lambda i: (i, 0)),
            pl.BlockSpec((_H, _NP), lambda i: (0, 0)),
            pl.BlockSpec((_BA, _H), lambda i: (i, 0)),
            pl.BlockSpec((_NP, _H * _OC), lambda i: (0, 0)),
            pl.BlockSpec((1, _H * _OC), lambda i: (0, 0)),
            pl.BlockSpec((_H * _OC, _C), lambda i: (0, 0)),
            pl.BlockSpec((_C, 1), lambda i: (0, 0)),
            pl.BlockSpec((_C, 1), lambda i: (0, 0)),
        ],
        out_specs=[
            pl.BlockSpec((_BA, _C), lambda i: (i, 0)),
            pl.BlockSpec((_BA, 1), lambda i: (i, 0)),
            pl.BlockSpec((_BA, 1), lambda i: (i, 0)),
        ],
        out_shape=[
            jax.ShapeDtypeStruct((_NP, _C), _F32),
            jax.ShapeDtypeStruct((_NP, 1), _F32),
            jax.ShapeDtypeStruct((_NP, 1), _F32),
        ],
    )(m, als.T, ald, h1, b1.reshape(1, -1), W2,
      att_src2.reshape(-1, 1), att_dst2.reshape(-1, 1))

    out = pl.pallas_call(
        _gat2_kernel,
        grid=(ga,),
        in_specs=[
            pl.BlockSpec((_BA, _NP), lambda i: (i, 0)),
            pl.BlockSpec((1, _NP), lambda i: (0, 0)),
            pl.BlockSpec((_BA, 1), lambda i: (i, 0)),
            pl.BlockSpec((_NP, _C), lambda i: (0, 0)),
            pl.BlockSpec((1, _C), lambda i: (0, 0)),
        ],
        out_specs=pl.BlockSpec((_BA, _C), lambda i: (i, 0)),
        out_shape=jax.ShapeDtypeStruct((_NP, _C), _F32),
    )(m, als2.T, ald2, h2, b2.reshape(1, -1))

    return out[:_N]


# speedup vs baseline: 1.1686x; 1.1686x over previous
import math

import jax
import jax.numpy as jnp
from jax.experimental import pallas as pl

_N = 10000
_D = 128
_H = 8
_OC = 8
_C = 40
_T = 5.0
_K = 10
_NP = 10240
_BD = 256
_BA = 128
_F32 = jnp.float32


def _dinv_kernel(m_ref, dinv_ref):
    m = m_ref[...].astype(_F32)
    deg = jnp.sum(m, axis=1, keepdims=True)
    dinv_ref[...] = jnp.where(deg > 0, jax.lax.rsqrt(deg), 0.0)


def _td_step_kernel(coef, m_ref, dinv_full_ref, dinv_blk_ref, cur_ref,
                    acc_ref, cur_out_ref, acc_out_ref):
    m = m_ref[...].astype(_F32)
    v = cur_ref[...] * dinv_full_ref[...]
    y = jnp.dot(m, v, preferred_element_type=_F32)
    cur_new = y * dinv_blk_ref[...]
    cur_out_ref[...] = cur_new
    acc_out_ref[...] = acc_ref[...] + coef * cur_new


def _proj1_kernel(acc_ref, w1_ref, smat_ref, dmat_ref,
                  h1_ref, als_ref, ald_ref):
    h1 = jnp.dot(acc_ref[...], w1_ref[...], preferred_element_type=_F32)
    h1_ref[...] = h1
    als_ref[...] = jnp.dot(h1, smat_ref[...], preferred_element_type=_F32)
    ald_ref[...] = jnp.dot(h1, dmat_ref[...], preferred_element_type=_F32)


def _gat1_kernel(m_ref, alst_ref, ald_ref, h1_ref, b1_ref, w2_ref,
                 s2_ref, d2_ref, h2_ref, als2_ref, ald2_ref):
    m = m_ref[...].astype(_F32)
    alst = alst_ref[...]
    ald = ald_ref[...]
    h1 = h1_ref[...]
    esp = jnp.exp(alst)
    esn = jnp.exp(0.2 * alst)
    edp = jnp.exp(ald)
    edn = jnp.exp(0.2 * ald)
    outs = []
    for h in range(_H):
        p = esp[h:h + 1, :] * edp[:, h:h + 1]
        n = esn[h:h + 1, :] * edn[:, h:h + 1]
        w = jnp.maximum(p, n) * m
        den = jnp.sum(w, axis=1, keepdims=True)
        num = jnp.dot(w, h1[:, h * _OC:(h + 1) * _OC],
                      preferred_element_type=_F32)
        outs.append(num / (den + 1e-16))
    z = jnp.concatenate(outs, axis=1) + b1_ref[...]
    z = jnp.where(z > 0, z, jnp.exp(jnp.minimum(z, 0.0)) - 1.0)
    h2 = jnp.dot(z, w2_ref[...], preferred_element_type=_F32)
    h2_ref[...] = h2
    als2_ref[...] = jnp.dot(h2, s2_ref[...], preferred_element_type=_F32)
    ald2_ref[...] = jnp.dot(h2, d2_ref[...], preferred_element_type=_F32)


def _gat2_kernel(m_ref, als2t_ref, ald2_ref, h2_ref, b2_ref, out_ref):
    m = m_ref[...].astype(_F32)
    als2t = als2t_ref[...]
    ald2 = ald2_ref[...]
    p = jnp.exp(als2t) * jnp.exp(ald2)
    n = jnp.exp(0.2 * als2t) * jnp.exp(0.2 * ald2)
    w = jnp.maximum(p, n) * m
    den = jnp.sum(w, axis=1, keepdims=True)
    num = jnp.dot(w, h2_ref[...], preferred_element_type=_F32)
    o = num / (den + 1e-16) + b2_ref[...]
    mx = jnp.max(o, axis=1, keepdims=True)
    l = o - mx
    lse = jnp.log(jnp.sum(jnp.exp(l), axis=1, keepdims=True))
    out_ref[...] = l - lse


def kernel(x, edge_index, W1, att_src1, att_dst1, b1, W2, att_src2,
           att_dst2, b2):
    loops = jnp.arange(_N, dtype=jnp.int32)
    src = jnp.concatenate([edge_index[0].astype(jnp.int32), loops])
    dst = jnp.concatenate([edge_index[1].astype(jnp.int32), loops])
    flat = dst * _NP + src
    m = jnp.zeros((_NP * _NP,), jnp.int8).at[flat].add(
        jnp.int8(1)).reshape(_NP, _NP)

    xp = jnp.pad(x, ((0, _NP - _N), (0, 0)))

    g = _NP // _BD
    dinv = pl.pallas_call(
        _dinv_kernel,
        grid=(g,),
        in_specs=[pl.BlockSpec((_BD, _NP), lambda i: (i, 0))],
        out_specs=pl.BlockSpec((_BD, 1), lambda i: (i, 0)),
        out_shape=jax.ShapeDtypeStruct((_NP, 1), _F32),
    )(m)

    coef = math.exp(-_T)
    acc = coef * xp
    cur = xp
    for k in range(1, _K + 1):
        coef = coef * _T / k
        step = pl.pallas_call(
            lambda mr, df, db, cr, ar, co, ao, _c=coef: _td_step_kernel(
                _c, mr, df, db, cr, ar, co, ao),
            grid=(g,),
            in_specs=[
                pl.BlockSpec((_BD, _NP), lambda i: (i, 0)),
                pl.BlockSpec((_NP, 1), lambda i: (0, 0)),
                pl.BlockSpec((_BD, 1), lambda i: (i, 0)),
                pl.BlockSpec((_NP, _D), lambda i: (0, 0)),
                pl.BlockSpec((_BD, _D), lambda i: (i, 0)),
            ],
            out_specs=[
                pl.BlockSpec((_BD, _D), lambda i: (i, 0)),
                pl.BlockSpec((_BD, _D), lambda i: (i, 0)),
            ],
            out_shape=[
                jax.ShapeDtypeStruct((_NP, _D), _F32),
                jax.ShapeDtypeStruct((_NP, _D), _F32),
            ],
        )
        cur, acc = step(m, dinv, dinv, cur, acc)

    smat = jnp.einsum('ho,hk->hok', att_src1,
                      jnp.eye(_H, dtype=_F32)).reshape(_H * _OC, _H)
    dmat = jnp.einsum('ho,hk->hok', att_dst1,
                      jnp.eye(_H, dtype=_F32)).reshape(_H * _OC, _H)

    h1, als, ald = pl.pallas_call(
        _proj1_kernel,
        grid=(g,),
        in_specs=[
            pl.BlockSpec((_BD, _D), lambda i: (i, 0)),
            pl.BlockSpec((_D, _H * _OC), lambda i: (0, 0)),
            pl.BlockSpec((_H * _OC, _H), lambda i: (0, 0)),
            pl.BlockSpec((_H * _OC, _H), lambda i: (0, 0)),
        ],
        out_specs=[
            pl.BlockSpec((_BD, _H * _OC), lambda i: (i, 0)),
            pl.BlockSpec((_BD, _H), lambda i: (i, 0)),
            pl.BlockSpec((_BD, _H), lambda i: (i, 0)),
        ],
        out_shape=[
            jax.ShapeDtypeStruct((_NP, _H * _OC), _F32),
            jax.ShapeDtypeStruct((_NP, _H), _F32),
            jax.ShapeDtypeStruct((_NP, _H), _F32),
        ],
    )(acc, W1, smat, dmat)

    ga = _NP // _BA
    h2, als2, ald2 = pl.pallas_call(
        _gat1_kernel,
        grid=(ga,),
        in_specs=[
            pl.BlockSpec((_BA, _NP), lambda i: (i, 0)),
            pl.BlockSpec((_H, _NP), lambda i: (0, 0)),
            pl.BlockSpec((_BA, _H), lambda i: (i, 0)),
            pl.BlockSpec((_NP, _H * _OC), lambda i: (0, 0)),
            pl.BlockSpec((1, _H * _OC), lambda i: (0, 0)),
            pl.BlockSpec((_H * _OC, _C), lambda i: (0, 0)),
            pl.BlockSpec((_C, 1), lambda i: (0, 0)),
            pl.BlockSpec((_C, 1), lambda i: (0, 0)),
        ],
        out_specs=[
            pl.BlockSpec((_BA, _C), lambda i: (i, 0)),
            pl.BlockSpec((_BA, 1), lambda i: (i, 0)),
            pl.BlockSpec((_BA, 1), lambda i: (i, 0)),
        ],
        out_shape=[
            jax.ShapeDtypeStruct((_NP, _C), _F32),
            jax.ShapeDtypeStruct((_NP, 1), _F32),
            jax.ShapeDtypeStruct((_NP, 1), _F32),
        ],
    )(m, als.T, ald, h1, b1.reshape(1, -1), W2,
      att_src2.reshape(-1, 1), att_dst2.reshape(-1, 1))

    out = pl.pallas_call(
        _gat2_kernel,
        grid=(ga,),
        in_specs=[
            pl.BlockSpec((_BA, _NP), lambda i: (i, 0)),
            pl.BlockSpec((1, _NP), lambda i: (0, 0)),
            pl.BlockSpec((_BA, 1), lambda i: (i, 0)),
            pl.BlockSpec((_NP, _C), lambda i: (0, 0)),
            pl.BlockSpec((1, _C), lambda i: (0, 0)),
        ],
        out_specs=pl.BlockSpec((_BA, _C), lambda i: (i, 0)),
        out_shape=jax.ShapeDtypeStruct((_NP, _C), _F32),
    )(m, als2.T, ald2, h2, b2.reshape(1, -1))

    return out[:_N]
